# baseline (speedup 1.0000x reference)
import functools

import jax
import jax.numpy as jnp
from jax import lax
from jax.experimental import pallas as pl
from jax.experimental.pallas import tpu as pltpu
from jax.experimental.pallas import tpu_sc as plsc

_NC = 2
_NS = 16
_NW = _NC * _NS
_LANES = 16
_CHUNK = 128
_ROW_W = 256


def _make_kernel(vocab, dim, batch):
    b_per_w = batch // _NW
    n_chunks = b_per_w // _CHUNK
    groups_per_chunk = _CHUNK // _LANES
    per_super = _ROW_W // dim
    mesh = plsc.VectorSubcoreMesh(core_axis_name="c", subcore_axis_name="s")

    @functools.partial(
        pl.kernel,
        out_type=jax.ShapeDtypeStruct((batch,), jnp.float32),
        mesh=mesh,
        compiler_params=pltpu.CompilerParams(needs_layout_passes=False),
        scratch_types=[
            pltpu.VMEM((n_chunks, _CHUNK), jnp.int32),
            pltpu.VMEM((n_chunks, _CHUNK), jnp.int32),
            pltpu.VMEM((b_per_w,), jnp.int32),
            pltpu.VMEM((_CHUNK, _ROW_W), jnp.float32),
            pltpu.VMEM((_CHUNK, _ROW_W), jnp.float32),
            pltpu.VMEM((b_per_w,), jnp.float32),
            pltpu.SemaphoreType.DMA,
        ],
    )
    def k(hero_sup, ctx_sup, off_in, hero_tab, ctx_tab, out,
          hsup_v, csup_v, off_v, hrow_v, crow_v, score_v, sem):
        wid = lax.axis_index("s") * _NC + lax.axis_index("c")
        base = wid * b_per_w

        pltpu.sync_copy(hero_sup.at[wid], hsup_v)
        pltpu.sync_copy(ctx_sup.at[wid], csup_v)
        pltpu.sync_copy(off_in.at[pl.ds(base, b_per_w)], off_v)

        lane = lax.iota(jnp.int32, _LANES)

        def chunk(ch, carry):
            e0 = ch * _CHUNK
            c1 = pltpu.async_copy(hero_tab.at[hsup_v.at[ch]], hrow_v, sem)
            c2 = pltpu.async_copy(ctx_tab.at[csup_v.at[ch]], crow_v, sem)
            c1.wait()
            c2.wait()

            def group(g, carry2):
                ge0 = g * _LANES
                off = off_v[pl.ds(e0 + ge0, _LANES)]
                hoff = off & 0xFFFF
                coff = lax.shift_right_logical(off, 16)
                row = ge0 + lane
                acc = jnp.zeros((_LANES,), jnp.float32)
                for d in range(dim):
                    h = plsc.load_gather(hrow_v, [row, hoff + d])
                    c = plsc.load_gather(crow_v, [row, coff + d])
                    acc = acc + h * c
                score_v[pl.ds(e0 + ge0, _LANES)] = acc
                return carry2

            lax.fori_loop(0, groups_per_chunk, group, 0)
            return carry

        lax.fori_loop(0, n_chunks, chunk, 0)

        pltpu.sync_copy(score_v, out.at[pl.ds(base, b_per_w)])

    return k


@jax.jit
def kernel(hero_ids, context_ids, hero_table, context_table):
    vocab, dim = hero_table.shape
    batch = hero_ids.shape[0]
    b_per_w = batch // _NW
    n_chunks = b_per_w // _CHUNK
    per_super = _ROW_W // dim
    k = _make_kernel(vocab, dim, batch)
    hids = hero_ids.astype(jnp.int32)
    cids = context_ids.astype(jnp.int32)
    hero_sup = (hids // per_super).reshape(_NW, n_chunks, _CHUNK)
    ctx_sup = (cids // per_super).reshape(_NW, n_chunks, _CHUNK)
    off = (hids % per_super) * dim + (((cids % per_super) * dim) << 16)
    hero_wide = hero_table.reshape(vocab // per_super, _ROW_W)
    ctx_wide = context_table.reshape(vocab // per_super, _ROW_W)
    return k(hero_sup, ctx_sup, off, hero_wide, ctx_wide)


# speedup vs baseline: 2.3933x; 2.3933x over previous
import functools

import jax
import jax.numpy as jnp
from jax import lax
from jax.experimental import pallas as pl
from jax.experimental.pallas import tpu as pltpu
from jax.experimental.pallas import tpu_sc as plsc

_NC = 2
_NS = 16
_NW = _NC * _NS
_LANES = 16


def _make_kernel(vocab, dim, batch):
    b_per_w = batch // _NW
    n_groups = b_per_w // _LANES
    mesh = plsc.VectorSubcoreMesh(core_axis_name="c", subcore_axis_name="s")

    @functools.partial(
        pl.kernel,
        out_type=jax.ShapeDtypeStruct((batch,), jnp.float32),
        mesh=mesh,
        compiler_params=pltpu.CompilerParams(needs_layout_passes=False),
        scratch_types=[
            pltpu.VMEM((b_per_w,), jnp.int32),
            pltpu.VMEM((b_per_w,), jnp.int32),
            pltpu.VMEM((_LANES, 8, dim), jnp.float32),
            pltpu.VMEM((_LANES, 8, dim), jnp.float32),
            pltpu.VMEM((_LANES, 8, dim), jnp.float32),
            pltpu.VMEM((_LANES, 8, dim), jnp.float32),
            pltpu.VMEM((b_per_w,), jnp.float32),
            pltpu.SemaphoreType.DMA,
            pltpu.SemaphoreType.DMA,
        ],
    )
    def k(hero_ids, ctx_ids, hero_tab, ctx_tab, out,
          hidx_v, cidx_v, hbuf_a, cbuf_a, hbuf_b, cbuf_b, score_v,
          sem_a, sem_b):
        wid = lax.axis_index("s") * _NC + lax.axis_index("c")
        base = wid * b_per_w

        pltpu.sync_copy(hero_ids.at[pl.ds(base, b_per_w)], hidx_v)
        pltpu.sync_copy(ctx_ids.at[pl.ds(base, b_per_w)], cidx_v)

        lane = lax.iota(jnp.int32, _LANES)

        def issue(g, hbuf, cbuf, sem):
            e0 = g * _LANES
            htile = lax.shift_right_logical(hidx_v[pl.ds(e0, _LANES)], 3)
            ctile = lax.shift_right_logical(cidx_v[pl.ds(e0, _LANES)], 3)
            for j in range(_LANES):
                pltpu.async_copy(hero_tab.at[htile[j]], hbuf.at[j], sem)
                pltpu.async_copy(ctx_tab.at[ctile[j]], cbuf.at[j], sem)

        def drain(hbuf, cbuf, sem):
            pltpu.make_async_copy(
                hero_tab.at[pl.ds(0, _LANES)], hbuf, sem).wait()
            pltpu.make_async_copy(
                ctx_tab.at[pl.ds(0, _LANES)], cbuf, sem).wait()

        def compute(g, hbuf, cbuf):
            e0 = g * _LANES
            hsub = hidx_v[pl.ds(e0, _LANES)] & 7
            csub = cidx_v[pl.ds(e0, _LANES)] & 7
            acc = jnp.zeros((_LANES,), jnp.float32)
            for d in range(dim):
                col = jnp.full((_LANES,), d, jnp.int32)
                h = plsc.load_gather(hbuf, [lane, hsub, col])
                c = plsc.load_gather(cbuf, [lane, csub, col])
                acc = acc + h * c
            score_v[pl.ds(e0, _LANES)] = acc

        issue(0, hbuf_a, cbuf_a, sem_a)

        def body(i, carry):
            g = 2 * i
            issue(g + 1, hbuf_b, cbuf_b, sem_b)
            drain(hbuf_a, cbuf_a, sem_a)
            compute(g, hbuf_a, cbuf_a)

            @pl.when(i < n_groups // 2 - 1)
            def _():
                issue(g + 2, hbuf_a, cbuf_a, sem_a)

            drain(hbuf_b, cbuf_b, sem_b)
            compute(g + 1, hbuf_b, cbuf_b)
            return carry

        lax.fori_loop(0, n_groups // 2, body, 0)

        pltpu.sync_copy(score_v, out.at[pl.ds(base, b_per_w)])

    return k


@jax.jit
def kernel(hero_ids, context_ids, hero_table, context_table):
    vocab, dim = hero_table.shape
    batch = hero_ids.shape[0]
    k = _make_kernel(vocab, dim, batch)
    hero3 = hero_table.reshape(vocab // 8, 8, dim)
    ctx3 = context_table.reshape(vocab // 8, 8, dim)
    return k(hero_ids.astype(jnp.int32), context_ids.astype(jnp.int32),
             hero3, ctx3)
